# baseline (speedup 1.0000x reference)
import functools

import numpy as np
import jax
import jax.numpy as jnp
from jax import lax
from jax.experimental import pallas as pl
from jax.experimental.pallas import tpu as pltpu
from jax.experimental.pallas import tpu_sc as plsc

N = 1_000_000
NS = 16
L = 16

T_TC = 802_816
GT = T_TC // 128
BR = 784
NSTEP = GT // BR
SUB = BR // 8

CH = 12_544
NPIECE = 2
PIECE = CH // NPIECE
UNROLL = 8
STEPS = PIECE // (L * UNROLL)
assert T_TC + (NS - 1) * CH < N <= T_TC + NS * CH
assert PIECE % (L * UNROLL) == 0 and (N - CH) % 8 == 0 and T_TC % 8 == 0


def _np_threefry2x32(k0, k1, x0, x1):
    def rotl(x, d):
        return ((x << np.uint32(d)) | (x >> np.uint32(32 - d))).astype(np.uint32)
    ks = [np.uint32(k0), np.uint32(k1),
          np.uint32(0x1BD11BDA) ^ np.uint32(k0) ^ np.uint32(k1)]
    x0 = (x0 + ks[0]).astype(np.uint32)
    x1 = (x1 + ks[1]).astype(np.uint32)
    rot = ((13, 15, 26, 6), (17, 29, 16, 24))
    for i in range(5):
        for r in rot[i % 2]:
            x0 = (x0 + x1).astype(np.uint32)
            x1 = rotl(x1, r) ^ x0
        x0 = (x0 + ks[(i + 1) % 3]).astype(np.uint32)
        x1 = (x1 + ks[(i + 2) % 3] + np.uint32(i + 1)).astype(np.uint32)
    return x0, x1


def _exp_gumbel_const(seed, n):
    idx = np.arange(n, dtype=np.uint64)
    hi = (idx >> np.uint64(32)).astype(np.uint32)
    lo = (idx & np.uint64(0xFFFFFFFF)).astype(np.uint32)
    o0, o1 = _np_threefry2x32(np.uint32(seed >> 32), np.uint32(seed & 0xFFFFFFFF),
                              hi, lo)
    bits = o0 ^ o1
    fb = ((bits >> np.uint32(9)) | np.uint32(0x3F800000)).view(np.float32)
    floats = (fb - np.float32(1.0)).astype(np.float32)
    tiny = np.float32(np.finfo(np.float32).tiny)
    span = np.float32(np.float32(1.0) - tiny)
    u = np.maximum(tiny, (floats * span + tiny).astype(np.float32))
    return (1.0 / (-np.log(u.astype(np.float64)))).astype(np.float32)


_EXP_GUMBEL = _exp_gumbel_const(42, N)
_EXP_GUMBEL_TC = _EXP_GUMBEL[:T_TC].reshape(GT, 128)

_I32MAX = np.int32(2**31 - 1)


def _sc_tail(nw_hbm, att_hbm, eg_hbm, bs_out, bi_out, ba_out,
             nw_v, at_v, eg_v, sc_v, si_v, sa_v, sems):
    wid = lax.axis_index("s")
    base = jnp.where(wid == NS - 1, N - CH, T_TC + wid * CH)
    base = pl.multiple_of(base, 8)
    copies = []
    for p in range(NPIECE):
        src = pl.ds(base + p * PIECE, PIECE)
        dst = pl.ds(p * PIECE, PIECE)
        copies.append(
            (pltpu.async_copy(nw_hbm.at[src], nw_v.at[dst], sems.at[p, 0]),
             pltpu.async_copy(att_hbm.at[src], at_v.at[dst], sems.at[p, 1]),
             pltpu.async_copy(eg_hbm.at[src], eg_v.at[dst], sems.at[p, 2])))
    idx0 = lax.iota(jnp.int32, L) + base

    acc = [(jnp.full((L,), -1.0, jnp.float32),
            jnp.zeros((L,), jnp.int32),
            jnp.zeros((L,), jnp.float32)) for _ in range(UNROLL)]

    for p in range(NPIECE):
        for c in copies[p]:
            c.wait()
        pbase = p * PIECE

        def body(k, carry):
            out = []
            for u in range(UNROLL):
                best, besti, besta = carry[u]
                off = pbase + k * (L * UNROLL) + u * L
                atv = at_v[pl.ds(off, L)]
                s = nw_v[pl.ds(off, L)] * atv * eg_v[pl.ds(off, L)]
                iv = idx0 + off
                m = s > best
                out.append((jnp.where(m, s, best),
                            jnp.where(m, iv, besti),
                            jnp.where(m, atv, besta)))
            return tuple(out)

        acc = lax.fori_loop(0, STEPS, body, tuple(acc))

    best, besti, besta = acc[0]
    for u in range(1, UNROLL):
        s, i, a = acc[u]
        take = (s > best) | ((s == best) & (i < besti))
        best = jnp.where(take, s, best)
        besti = jnp.where(take, i, besti)
        besta = jnp.where(take, a, besta)
    sc_v[...] = best
    si_v[...] = besti
    sa_v[...] = besta
    pltpu.sync_copy(sc_v, bs_out.at[wid])
    pltpu.sync_copy(si_v, bi_out.at[wid])
    pltpu.sync_copy(sa_v, ba_out.at[wid])


@functools.cache
def _get_sc_tail():
    return pl.kernel(
        _sc_tail,
        out_type=(jax.ShapeDtypeStruct((NS, L), jnp.float32),
                  jax.ShapeDtypeStruct((NS, L), jnp.int32),
                  jax.ShapeDtypeStruct((NS, L), jnp.float32)),
        mesh=plsc.VectorSubcoreMesh(core_axis_name="c", subcore_axis_name="s",
                                    num_cores=1, num_subcores=NS),
        scratch_types=[pltpu.VMEM((CH,), jnp.float32),
                       pltpu.VMEM((CH,), jnp.float32),
                       pltpu.VMEM((CH,), jnp.float32),
                       pltpu.VMEM((L,), jnp.float32),
                       pltpu.VMEM((L,), jnp.int32),
                       pltpu.VMEM((L,), jnp.float32),
                       pltpu.SemaphoreType.DMA((NPIECE, 3))],
    )


def _tc_front(nw_ref, att_ref, eg_ref, s_out, n_out, a_out,
              acc_s, acc_i, acc_a):
    g = pl.program_id(0)

    @pl.when(g == 0)
    def _():
        acc_s[...] = jnp.full((8, 128), -1.0, jnp.float32)
        acc_i[...] = jnp.zeros((8, 128), jnp.int32)
        acc_a[...] = jnp.zeros((8, 128), jnp.float32)

    iota2d = (lax.broadcasted_iota(jnp.int32, (8, 128), 0) * 128
              + lax.broadcasted_iota(jnp.int32, (8, 128), 1))
    best = acc_s[...]
    besti = acc_i[...]
    besta = acc_a[...]
    for t in range(SUB):
        rows = pl.ds(8 * t, 8)
        atv = att_ref[rows, :]
        x = nw_ref[rows, :] * atv * eg_ref[rows, :]
        iv = iota2d + (g * BR + 8 * t) * 128
        m = x > best
        best = jnp.where(m, x, best)
        besti = jnp.where(m, iv, besti)
        besta = jnp.where(m, atv, besta)
    acc_s[...] = best
    acc_i[...] = besti
    acc_a[...] = besta

    @pl.when(g == NSTEP - 1)
    def _():
        mx = jnp.max(best)
        hit = best == mx
        node = jnp.min(jnp.where(hit, besti, _I32MAX))
        att = jnp.max(jnp.where(hit & (besti == node), besta,
                                jnp.float32(-1.0)))
        s_out[0, 0] = mx
        n_out[0, 0] = node
        a_out[0, 0] = att


@functools.cache
def _get_tc_front():
    return pl.pallas_call(
        _tc_front,
        grid=(NSTEP,),
        in_specs=[pl.BlockSpec((BR, 128), lambda g: (g, 0)),
                  pl.BlockSpec((BR, 128), lambda g: (g, 0)),
                  pl.BlockSpec((BR, 128), lambda g: (g, 0))],
        out_specs=(pl.BlockSpec(memory_space=pltpu.SMEM),
                   pl.BlockSpec(memory_space=pltpu.SMEM),
                   pl.BlockSpec(memory_space=pltpu.SMEM)),
        out_shape=(jax.ShapeDtypeStruct((1, 1), jnp.float32),
                   jax.ShapeDtypeStruct((1, 1), jnp.int32),
                   jax.ShapeDtypeStruct((1, 1), jnp.float32)),
        scratch_shapes=[pltpu.VMEM((8, 128), jnp.float32),
                        pltpu.VMEM((8, 128), jnp.int32),
                        pltpu.VMEM((8, 128), jnp.float32)],
        compiler_params=pltpu.CompilerParams(
            dimension_semantics=("arbitrary",)),
    )


def _merge_body(bs_ref, bi_ref, ba_ref, stc_ref, ntc_ref, atc_ref,
                node_ref, att_ref):
    s = bs_ref[...]
    i = bi_ref[...]
    a = ba_ref[...]
    stc = stc_ref[0, 0]
    ntc = ntc_ref[0, 0]
    atc = atc_ref[0, 0]
    m = jnp.maximum(jnp.max(s), stc)
    hit = s == m
    node = jnp.min(jnp.where(hit, i, _I32MAX))
    node = jnp.where(stc == m, jnp.minimum(node, ntc), node)
    att = jnp.max(jnp.where(hit & (i == node), a, jnp.float32(-1.0)))
    att = jnp.where((stc == m) & (ntc == node), jnp.maximum(att, atc), att)
    node_ref[0, 0] = node
    att_ref[0, 0] = att


@functools.cache
def _get_merge():
    return pl.pallas_call(
        _merge_body,
        in_specs=[pl.BlockSpec((NS, L), lambda: (0, 0))] * 3
        + [pl.BlockSpec(memory_space=pltpu.SMEM)] * 3,
        out_shape=(jax.ShapeDtypeStruct((1, 1), jnp.int32),
                   jax.ShapeDtypeStruct((1, 1), jnp.float32)),
        out_specs=(pl.BlockSpec(memory_space=pltpu.SMEM),
                   pl.BlockSpec(memory_space=pltpu.SMEM)),
    )


def kernel(neighbor_weights, attention):
    eg_sc = jnp.asarray(_EXP_GUMBEL)
    eg_tc = jnp.asarray(_EXP_GUMBEL_TC)
    bs, bi, ba = _get_sc_tail()(neighbor_weights, attention, eg_sc)
    nw2 = neighbor_weights[:T_TC].reshape(GT, 128)
    at2 = attention[:T_TC].reshape(GT, 128)
    stc, ntc, atc = _get_tc_front()(nw2, at2, eg_tc)
    node, att = _get_merge()(bs, bi, ba, stc, ntc, atc)
    return node[0, 0], att[0, 0]
